# baseline (speedup 1.0000x reference)
import jax
import jax.numpy as jnp
import numpy as np
from jax.experimental import pallas as pl

B = 2048
N = 5
D = 64
H = 64
A = 16
PAIRS = [(i, j) for i in range(N) for j in range(N) if i != j]
P = len(PAIRS)

BB = 512


def _gnn_kernel(states_ref, act_ref, Web_ref, beb_ref, Wns_ref, Wna_ref,
                Wnh_ref, bn_ref, out_ref):
    s = states_ref[...]
    a = act_ref[...]
    s_bf = s.astype(jnp.bfloat16)
    a_bf = a.astype(jnp.bfloat16)

    msgs = jnp.tanh(
        jnp.dot(s_bf, Web_ref[...], preferred_element_type=jnp.float32)
        + beb_ref[...]
    )

    agg = jnp.concatenate(
        [msgs[:, (4 * n) * H:(4 * n + 1) * H]
         + msgs[:, (4 * n + 1) * H:(4 * n + 2) * H]
         + msgs[:, (4 * n + 2) * H:(4 * n + 3) * H]
         + msgs[:, (4 * n + 3) * H:(4 * n + 4) * H] for n in range(N)],
        axis=1,
    )

    pre = (
        jnp.dot(s_bf, Wns_ref[...], preferred_element_type=jnp.float32)
        + jnp.dot(a_bf, Wna_ref[...], preferred_element_type=jnp.float32)
        + jnp.dot(agg.astype(jnp.bfloat16), Wnh_ref[...],
                  preferred_element_type=jnp.float32)
        + bn_ref[...]
    )
    out_ref[...] = jnp.tanh(pre)


def _pack_weights(W_edge, b_edge, W_node, b_node):
    src = np.zeros((N, P), dtype=np.float32)
    dst = np.zeros((N, P), dtype=np.float32)
    for p, (i, j) in enumerate(PAIRS):
        src[i, p] = 1.0
        dst[j, p] = 1.0
    web = (jnp.einsum('np,pdh->ndph', jnp.asarray(src), W_edge[:, :D, :])
           + jnp.einsum('np,pdh->ndph', jnp.asarray(dst), W_edge[:, D:, :]))
    web = web.reshape(N * D, P * H)
    beb = b_edge.reshape(1, P * H)
    eye = jnp.eye(N, dtype=jnp.float32)
    wns = jnp.einsum('nm,ndo->ndmo', eye, W_node[:, :D, :]).reshape(N * D, N * D)
    wna = jnp.einsum('nm,ndo->ndmo', eye, W_node[:, D:D + A, :]).reshape(N * A, N * D)
    wnh = jnp.einsum('nm,ndo->ndmo', eye, W_node[:, D + A:, :]).reshape(N * H, N * D)
    bnb = b_node.reshape(1, N * D)
    bf = jnp.bfloat16
    return web.astype(bf), beb, wns.astype(bf), wna.astype(bf), wnh.astype(bf), bnb


def kernel(states, action_vec, W_edge, b_edge, W_node, b_node):
    s2 = states.reshape(B, N * D)
    a2 = action_vec.reshape(B, N * A)
    web, beb, wns, wna, wnh, bnb = _pack_weights(W_edge, b_edge, W_node, b_node)
    grid = (B // BB,)
    full = lambda *shape: None
    out = pl.pallas_call(
        _gnn_kernel,
        grid=grid,
        in_specs=[
            pl.BlockSpec((BB, N * D), lambda g: (g, 0)),
            pl.BlockSpec((BB, N * A), lambda g: (g, 0)),
            pl.BlockSpec((N * D, P * H), lambda g: (0, 0)),
            pl.BlockSpec((1, P * H), lambda g: (0, 0)),
            pl.BlockSpec((N * D, N * D), lambda g: (0, 0)),
            pl.BlockSpec((N * A, N * D), lambda g: (0, 0)),
            pl.BlockSpec((N * H, N * D), lambda g: (0, 0)),
            pl.BlockSpec((1, N * D), lambda g: (0, 0)),
        ],
        out_specs=pl.BlockSpec((BB, N * D), lambda g: (g, 0)),
        out_shape=jax.ShapeDtypeStruct((B, N * D), jnp.float32),
    )(s2, a2, web, beb, wns, wna, wnh, bnb)
    return out.reshape(B, N, D)


# speedup vs baseline: 1.3940x; 1.3940x over previous
import jax
import jax.numpy as jnp
from jax.experimental import pallas as pl
from jax.experimental import pallas as pl_mod
from jax.experimental.pallas import tpu as pltpu

B = 2048
N = 5
D = 64
H = 64
A = 16
PAIRS = [(i, j) for i in range(N) for j in range(N) if i != j]
P = len(PAIRS)

BB = 512


def _gnn_kernel(states_ref, act_ref, We_ref, be_ref, Wn_ref, bn_ref, out_ref,
                We_s, Wn_s):
    g = pl.program_id(0)

    @pl.when(g == 0)
    def _cast_weights():
        We_s[...] = We_ref[...].astype(jnp.bfloat16)
        Wn_s[...] = Wn_ref[...].astype(jnp.bfloat16)

    s = states_ref[...]
    a = act_ref[...]
    s_bf = s.astype(jnp.bfloat16)

    agg = [None] * N
    for p, (i, j) in enumerate(PAIRS):
        edge_in = jnp.concatenate(
            [s_bf[:, i * D:(i + 1) * D], s_bf[:, j * D:(j + 1) * D]], axis=1)
        m = jnp.tanh(
            jnp.dot(edge_in, We_s[p], preferred_element_type=jnp.float32)
            + be_ref[p]
        )
        agg[i] = m if agg[i] is None else agg[i] + m

    a_bf = a.astype(jnp.bfloat16)
    for n in range(N):
        node_in = jnp.concatenate(
            [s_bf[:, n * D:(n + 1) * D], a_bf[:, n * A:(n + 1) * A],
             agg[n].astype(jnp.bfloat16)], axis=1)
        o = jnp.tanh(
            jnp.dot(node_in, Wn_s[n], preferred_element_type=jnp.float32)
            + bn_ref[n]
        )
        out_ref[:, n * D:(n + 1) * D] = o


def kernel(states, action_vec, W_edge, b_edge, W_node, b_node):
    s2 = states.reshape(B, N * D)
    a2 = action_vec.reshape(B, N * A)
    grid = (B // BB,)
    out = pl.pallas_call(
        _gnn_kernel,
        grid=grid,
        in_specs=[
            pl.BlockSpec((BB, N * D), lambda g: (g, 0)),
            pl.BlockSpec((BB, N * A), lambda g: (g, 0)),
            pl.BlockSpec((P, 2 * D, H), lambda g: (0, 0, 0)),
            pl.BlockSpec((P, H), lambda g: (0, 0)),
            pl.BlockSpec((N, D + A + H, D), lambda g: (0, 0, 0)),
            pl.BlockSpec((N, D), lambda g: (0, 0)),
        ],
        out_specs=pl.BlockSpec((BB, N * D), lambda g: (g, 0)),
        out_shape=jax.ShapeDtypeStruct((B, N * D), jnp.float32),
        scratch_shapes=[
            pltpu.VMEM((P, 2 * D, H), jnp.bfloat16),
            pltpu.VMEM((N, D + A + H, D), jnp.bfloat16),
        ],
    )(s2, a2, W_edge, b_edge, W_node, b_node)
    return out.reshape(B, N, D)
